# baseline (speedup 1.0000x reference)
import functools

import jax
import jax.numpy as jnp
from jax import lax
from jax.experimental import pallas as pl
from jax.experimental.pallas import tpu as pltpu
from jax.experimental.pallas import tpu_sc as plsc

TOPK = 8
NE = 64
H = 1024
NC = 2
NS = 16
NW = NC * NS
CHUNKS = 4


def _logits_kernel(x_ref, w_ref, out_ref):
    out_ref[0] = lax.dot_general(
        w_ref[...], x_ref[...], (((1,), (1,)), ((), ())),
        preferred_element_type=jnp.float32,
    )


def _tc_logits_chunk(x, weight, tpc, chunk):
    return pl.pallas_call(
        _logits_kernel,
        grid=(NW,),
        in_specs=[
            pl.BlockSpec((tpc, H), lambda i, c=chunk: (c * NW + i, 0)),
            pl.BlockSpec((NE, H), lambda i: (0, 0)),
        ],
        out_specs=pl.BlockSpec((1, NE, tpc), lambda i: (i, 0, 0)),
        out_shape=jax.ShapeDtypeStruct((NW, NE, tpc), jnp.float32),
    )(x, weight)


def _make_sc_topk_body(tpc):
    def _sc_topk_body(lg_hbm, idx_hbm, wgt_hbm, lg_v, idx_v, wgt_v):
        wid = lax.axis_index("s") * NC + lax.axis_index("c")
        pltpu.sync_copy(lg_hbm.at[wid], lg_v)

        def group(g, carry):
            t0 = pl.multiple_of(g * 16, 16)
            neg = jnp.full((16,), -jnp.inf, jnp.float32)
            zero = jnp.zeros((16,), jnp.int32)
            vs = [neg] * TOPK
            ixs = [zero] * TOPK
            for e in range(NE):
                x = lg_v[e, pl.ds(t0, 16)]
                ev = jnp.full((16,), e, jnp.int32)
                gt = [x > vs[j] for j in range(TOPK)]
                nv = [None] * TOPK
                ni = [None] * TOPK
                nv[0] = jnp.where(gt[0], x, vs[0])
                ni[0] = jnp.where(gt[0], ev, ixs[0])
                for j in range(1, TOPK):
                    nv[j] = jnp.where(gt[j], jnp.where(gt[j - 1], vs[j - 1], x), vs[j])
                    ni[j] = jnp.where(gt[j], jnp.where(gt[j - 1], ixs[j - 1], ev), ixs[j])
                vs, ixs = nv, ni
            m = vs[0]
            es = [jnp.exp(vs[k] - m) for k in range(TOPK)]
            s = es[0]
            for k in range(1, TOPK):
                s = s + es[k]
            r = 1.0 / s
            for k in range(TOPK):
                idx_v[k, pl.ds(t0, 16)] = ixs[k]
                wgt_v[k, pl.ds(t0, 16)] = es[k] * r
            return carry

        lax.fori_loop(0, tpc // 16, group, 0)
        pltpu.sync_copy(idx_v, idx_hbm.at[wid])
        pltpu.sync_copy(wgt_v, wgt_hbm.at[wid])

    return _sc_topk_body


def _sc_topk_chunk(logits3, tpc):
    mesh = plsc.VectorSubcoreMesh(core_axis_name="c", subcore_axis_name="s")
    fn = pl.kernel(
        _make_sc_topk_body(tpc),
        out_type=[
            jax.ShapeDtypeStruct((NW, TOPK, tpc), jnp.int32),
            jax.ShapeDtypeStruct((NW, TOPK, tpc), jnp.float32),
        ],
        mesh=mesh,
        scratch_types=[
            pltpu.VMEM((NE, tpc), jnp.float32),
            pltpu.VMEM((TOPK, tpc), jnp.int32),
            pltpu.VMEM((TOPK, tpc), jnp.float32),
        ],
    )
    return fn(logits3)


def kernel(hidden_states, weight):
    x = hidden_states.reshape(-1, hidden_states.shape[-1])
    t = x.shape[0]
    tpc = t // (CHUNKS * NW)
    idx_parts = [None] * CHUNKS
    wgt_parts = [None] * CHUNKS
    logits_parts = [None] * CHUNKS
    logits_parts[0] = _tc_logits_chunk(x, weight, tpc, 0)
    for c in range(1, CHUNKS):
        logits_parts[c] = _tc_logits_chunk(x, weight, tpc, c)
        idx_parts[c - 1], wgt_parts[c - 1] = _sc_topk_chunk(logits_parts[c - 1], tpc)
    idx_parts[-1], wgt_parts[-1] = _sc_topk_chunk(logits_parts[-1], tpc)
    idx3 = jnp.concatenate(idx_parts, axis=0)
    wgt3 = jnp.concatenate(wgt_parts, axis=0)
    idx = idx3.transpose(0, 2, 1).reshape(t, TOPK)
    wgt = wgt3.transpose(0, 2, 1).reshape(t, TOPK)
    return idx, wgt


# speedup vs baseline: 1.0022x; 1.0022x over previous
import functools

import jax
import jax.numpy as jnp
from jax import lax
from jax.experimental import pallas as pl
from jax.experimental.pallas import tpu as pltpu
from jax.experimental.pallas import tpu_sc as plsc

TOPK = 8
NE = 64
H = 1024
NC = 2
NS = 16
NW = NC * NS
CHUNKS = 4


def _logits_kernel(x_ref, w_ref, out_ref):
    out_ref[0] = lax.dot_general(
        w_ref[...], x_ref[...], (((1,), (1,)), ((), ())),
        preferred_element_type=jnp.float32,
    )


def _tc_logits_chunk(x, weight, tpc, chunk):
    return pl.pallas_call(
        _logits_kernel,
        grid=(NW,),
        in_specs=[
            pl.BlockSpec((tpc, H), lambda i, c=chunk: (c * NW + i, 0)),
            pl.BlockSpec((NE, H), lambda i: (0, 0)),
        ],
        out_specs=pl.BlockSpec((1, NE, tpc), lambda i: (i, 0, 0)),
        out_shape=jax.ShapeDtypeStruct((NW, NE, tpc), jnp.float32),
        cost_estimate=pl.CostEstimate(
            flops=2 * NW * tpc * H * NE,
            transcendentals=0,
            bytes_accessed=NW * tpc * H * 4 + NE * H * 4 + NW * NE * tpc * 4,
        ),
    )(x, weight)


def _make_sc_topk_body(tpc):
    def _sc_topk_body(lg_hbm, idx_hbm, wgt_hbm, lg_v, idx_v, wgt_v):
        wid = lax.axis_index("s") * NC + lax.axis_index("c")
        pltpu.sync_copy(lg_hbm.at[wid], lg_v)

        def group(g, carry):
            t0 = pl.multiple_of(g * 16, 16)
            neg = jnp.full((16,), -jnp.inf, jnp.float32)
            zero = jnp.zeros((16,), jnp.int32)
            vs = [neg] * TOPK
            ixs = [zero] * TOPK
            for e in range(NE):
                x = lg_v[e, pl.ds(t0, 16)]
                ev = jnp.full((16,), e, jnp.int32)
                gt = [x > vs[j] for j in range(TOPK)]
                nv = [None] * TOPK
                ni = [None] * TOPK
                nv[0] = jnp.where(gt[0], x, vs[0])
                ni[0] = jnp.where(gt[0], ev, ixs[0])
                for j in range(1, TOPK):
                    nv[j] = jnp.where(gt[j], jnp.where(gt[j - 1], vs[j - 1], x), vs[j])
                    ni[j] = jnp.where(gt[j], jnp.where(gt[j - 1], ixs[j - 1], ev), ixs[j])
                vs, ixs = nv, ni
            m = vs[0]
            es = [jnp.exp(vs[k] - m) for k in range(TOPK)]
            s = es[0]
            for k in range(1, TOPK):
                s = s + es[k]
            r = 1.0 / s
            for k in range(TOPK):
                idx_v[k, pl.ds(t0, 16)] = ixs[k]
                wgt_v[k, pl.ds(t0, 16)] = es[k] * r
            return carry

        lax.fori_loop(0, tpc // 16, group, 0)
        pltpu.sync_copy(idx_v, idx_hbm.at[wid])
        pltpu.sync_copy(wgt_v, wgt_hbm.at[wid])

    return _sc_topk_body


def _sc_topk_chunk(logits3, tpc):
    mesh = plsc.VectorSubcoreMesh(core_axis_name="c", subcore_axis_name="s")
    fn = pl.kernel(
        _make_sc_topk_body(tpc),
        out_type=[
            jax.ShapeDtypeStruct((NW, TOPK, tpc), jnp.int32),
            jax.ShapeDtypeStruct((NW, TOPK, tpc), jnp.float32),
        ],
        mesh=mesh,
        scratch_types=[
            pltpu.VMEM((NE, tpc), jnp.float32),
            pltpu.VMEM((TOPK, tpc), jnp.int32),
            pltpu.VMEM((TOPK, tpc), jnp.float32),
        ],
        cost_estimate=pl.CostEstimate(
            flops=50 * NW * NE * tpc,
            transcendentals=NW * TOPK * tpc,
            bytes_accessed=NW * NE * tpc * 4 + 2 * NW * TOPK * tpc * 4,
        ),
    )
    return fn(logits3)


def kernel(hidden_states, weight):
    x = hidden_states.reshape(-1, hidden_states.shape[-1])
    t = x.shape[0]
    tpc = t // (CHUNKS * NW)
    idx_parts = [None] * CHUNKS
    wgt_parts = [None] * CHUNKS
    logits_parts = [None] * CHUNKS
    logits_parts[0] = _tc_logits_chunk(x, weight, tpc, 0)
    for c in range(1, CHUNKS):
        logits_parts[c] = _tc_logits_chunk(x, weight, tpc, c)
        idx_parts[c - 1], wgt_parts[c - 1] = _sc_topk_chunk(logits_parts[c - 1], tpc)
    idx_parts[-1], wgt_parts[-1] = _sc_topk_chunk(logits_parts[-1], tpc)
    idx3 = jnp.concatenate(idx_parts, axis=0)
    wgt3 = jnp.concatenate(wgt_parts, axis=0)
    idx = idx3.transpose(0, 2, 1).reshape(t, TOPK)
    wgt = wgt3.transpose(0, 2, 1).reshape(t, TOPK)
    return idx, wgt


# speedup vs baseline: 1.3379x; 1.3350x over previous
import functools

import jax
import jax.numpy as jnp
from jax import lax
from jax.experimental import pallas as pl
from jax.experimental.pallas import tpu as pltpu
from jax.experimental.pallas import tpu_sc as plsc

TOPK = 8
NE = 64
H = 1024
NC = 2
NS = 16
NW = NC * NS
CHUNKS = 1


def _logits_kernel(x_ref, w_ref, out_ref):
    out_ref[0] = lax.dot_general(
        w_ref[...], x_ref[...], (((1,), (1,)), ((), ())),
        preferred_element_type=jnp.float32,
    )


def _tc_logits_chunk(x, weight, tpc, chunk):
    return pl.pallas_call(
        _logits_kernel,
        grid=(NW,),
        in_specs=[
            pl.BlockSpec((tpc, H), lambda i, c=chunk: (c * NW + i, 0)),
            pl.BlockSpec((NE, H), lambda i: (0, 0)),
        ],
        out_specs=pl.BlockSpec((1, NE, tpc), lambda i: (i, 0, 0)),
        out_shape=jax.ShapeDtypeStruct((NW, NE, tpc), jnp.float32),
        cost_estimate=pl.CostEstimate(
            flops=2 * NW * tpc * H * NE,
            transcendentals=0,
            bytes_accessed=NW * tpc * H * 4 + NE * H * 4 + NW * NE * tpc * 4,
        ),
    )(x, weight)


def _make_sc_topk_body(tpc):
    def _sc_topk_body(lg_hbm, idx_hbm, wgt_hbm, lg_v, idx_v, wgt_v):
        wid = lax.axis_index("s") * NC + lax.axis_index("c")
        pltpu.sync_copy(lg_hbm.at[wid], lg_v)

        def group(g, carry):
            t0 = pl.multiple_of(g * 16, 16)
            neg = jnp.full((16,), -jnp.inf, jnp.float32)
            zero = jnp.zeros((16,), jnp.int32)
            vs = [neg] * TOPK
            ixs = [zero] * TOPK
            for e in range(NE):
                x = lg_v[e, pl.ds(t0, 16)]
                ev = jnp.full((16,), e, jnp.int32)
                gt = [x > vs[j] for j in range(TOPK)]
                nv = [None] * TOPK
                ni = [None] * TOPK
                nv[0] = jnp.where(gt[0], x, vs[0])
                ni[0] = jnp.where(gt[0], ev, ixs[0])
                for j in range(1, TOPK):
                    nv[j] = jnp.where(gt[j], jnp.where(gt[j - 1], vs[j - 1], x), vs[j])
                    ni[j] = jnp.where(gt[j], jnp.where(gt[j - 1], ixs[j - 1], ev), ixs[j])
                vs, ixs = nv, ni
            m = vs[0]
            es = [jnp.exp(vs[k] - m) for k in range(TOPK)]
            s = es[0]
            for k in range(1, TOPK):
                s = s + es[k]
            r = 1.0 / s
            for k in range(TOPK):
                idx_v[k, pl.ds(t0, 16)] = ixs[k]
                wgt_v[k, pl.ds(t0, 16)] = es[k] * r
            return carry

        lax.fori_loop(0, tpc // 16, group, 0)
        pltpu.sync_copy(idx_v, idx_hbm.at[wid])
        pltpu.sync_copy(wgt_v, wgt_hbm.at[wid])

    return _sc_topk_body


def _sc_topk_chunk(logits3, tpc):
    mesh = plsc.VectorSubcoreMesh(core_axis_name="c", subcore_axis_name="s")
    fn = pl.kernel(
        _make_sc_topk_body(tpc),
        out_type=[
            jax.ShapeDtypeStruct((NW, TOPK, tpc), jnp.int32),
            jax.ShapeDtypeStruct((NW, TOPK, tpc), jnp.float32),
        ],
        mesh=mesh,
        scratch_types=[
            pltpu.VMEM((NE, tpc), jnp.float32),
            pltpu.VMEM((TOPK, tpc), jnp.int32),
            pltpu.VMEM((TOPK, tpc), jnp.float32),
        ],
        cost_estimate=pl.CostEstimate(
            flops=50 * NW * NE * tpc,
            transcendentals=NW * TOPK * tpc,
            bytes_accessed=NW * NE * tpc * 4 + 2 * NW * TOPK * tpc * 4,
        ),
    )
    return fn(logits3)


def kernel(hidden_states, weight):
    x = hidden_states.reshape(-1, hidden_states.shape[-1])
    t = x.shape[0]
    tpc = t // (CHUNKS * NW)
    idx_parts = [None] * CHUNKS
    wgt_parts = [None] * CHUNKS
    logits_parts = [None] * CHUNKS
    logits_parts[0] = _tc_logits_chunk(x, weight, tpc, 0)
    for c in range(1, CHUNKS):
        logits_parts[c] = _tc_logits_chunk(x, weight, tpc, c)
        idx_parts[c - 1], wgt_parts[c - 1] = _sc_topk_chunk(logits_parts[c - 1], tpc)
    idx_parts[-1], wgt_parts[-1] = _sc_topk_chunk(logits_parts[-1], tpc)
    idx3 = jnp.concatenate(idx_parts, axis=0)
    wgt3 = jnp.concatenate(wgt_parts, axis=0)
    idx = idx3.transpose(0, 2, 1).reshape(t, TOPK)
    wgt = wgt3.transpose(0, 2, 1).reshape(t, TOPK)
    return idx, wgt


# speedup vs baseline: 1.4507x; 1.0843x over previous
import functools

import jax
import jax.numpy as jnp
from jax import lax
from jax.experimental import pallas as pl
from jax.experimental.pallas import tpu as pltpu
from jax.experimental.pallas import tpu_sc as plsc

TOPK = 8
NE = 64
H = 1024
NC = 2
NS = 16
NW = NC * NS
CHUNKS = 1


TC_FAN = 2


def _logits_kernel(x_ref, w_ref, out_ref):
    w = w_ref[...]
    tpc = out_ref.shape[2]
    for j in range(TC_FAN):
        out_ref[j] = lax.dot_general(
            w, x_ref[pl.ds(j * tpc, tpc), :], (((1,), (1,)), ((), ())),
            preferred_element_type=jnp.float32,
        )


def _tc_logits_chunk(x, weight, tpc, chunk):
    return pl.pallas_call(
        _logits_kernel,
        grid=(NW // TC_FAN,),
        in_specs=[
            pl.BlockSpec((TC_FAN * tpc, H), lambda i, c=chunk: (c * NW // TC_FAN + i, 0)),
            pl.BlockSpec((NE, H), lambda i: (0, 0)),
        ],
        out_specs=pl.BlockSpec((TC_FAN, NE, tpc), lambda i: (i, 0, 0)),
        out_shape=jax.ShapeDtypeStruct((NW, NE, tpc), jnp.float32),
        cost_estimate=pl.CostEstimate(
            flops=2 * NW * tpc * H * NE,
            transcendentals=0,
            bytes_accessed=NW * tpc * H * 4 + NE * H * 4 + NW * NE * tpc * 4,
        ),
    )(x, weight)


def _make_sc_topk_body(tpc):
    def _sc_topk_body(lg_hbm, idx_hbm, wgt_hbm, lg_v, idx_v, wgt_v):
        wid = lax.axis_index("s") * NC + lax.axis_index("c")
        pltpu.sync_copy(lg_hbm.at[wid], lg_v)

        def group(g, carry):
            t0 = pl.multiple_of(g * 16, 16)
            neg = jnp.full((16,), -jnp.inf, jnp.float32)
            zero = jnp.zeros((16,), jnp.int32)
            vs = [neg] * TOPK
            ixs = [zero] * TOPK
            for e in range(NE):
                x = lg_v[e, pl.ds(t0, 16)]
                ev = jnp.full((16,), e, jnp.int32)
                gt = [x > vs[j] for j in range(TOPK)]
                nv = [None] * TOPK
                ni = [None] * TOPK
                nv[0] = jnp.where(gt[0], x, vs[0])
                ni[0] = jnp.where(gt[0], ev, ixs[0])
                for j in range(1, TOPK):
                    nv[j] = jnp.where(gt[j], jnp.where(gt[j - 1], vs[j - 1], x), vs[j])
                    ni[j] = jnp.where(gt[j], jnp.where(gt[j - 1], ixs[j - 1], ev), ixs[j])
                vs, ixs = nv, ni
            m = vs[0]
            es = [jnp.exp(vs[k] - m) for k in range(TOPK)]
            s = es[0]
            for k in range(1, TOPK):
                s = s + es[k]
            r = 1.0 / s
            for k in range(TOPK):
                idx_v[k, pl.ds(t0, 16)] = ixs[k]
                wgt_v[k, pl.ds(t0, 16)] = es[k] * r
            return carry

        lax.fori_loop(0, tpc // 16, group, 0)
        pltpu.sync_copy(idx_v, idx_hbm.at[wid])
        pltpu.sync_copy(wgt_v, wgt_hbm.at[wid])

    return _sc_topk_body


def _sc_topk_chunk(logits3, tpc):
    mesh = plsc.VectorSubcoreMesh(core_axis_name="c", subcore_axis_name="s")
    fn = pl.kernel(
        _make_sc_topk_body(tpc),
        out_type=[
            jax.ShapeDtypeStruct((NW, TOPK, tpc), jnp.int32),
            jax.ShapeDtypeStruct((NW, TOPK, tpc), jnp.float32),
        ],
        mesh=mesh,
        scratch_types=[
            pltpu.VMEM((NE, tpc), jnp.float32),
            pltpu.VMEM((TOPK, tpc), jnp.int32),
            pltpu.VMEM((TOPK, tpc), jnp.float32),
        ],
        cost_estimate=pl.CostEstimate(
            flops=50 * NW * NE * tpc,
            transcendentals=NW * TOPK * tpc,
            bytes_accessed=NW * NE * tpc * 4 + 2 * NW * TOPK * tpc * 4,
        ),
    )
    return fn(logits3)


def kernel(hidden_states, weight):
    x = hidden_states.reshape(-1, hidden_states.shape[-1])
    t = x.shape[0]
    tpc = t // (CHUNKS * NW)
    idx_parts = [None] * CHUNKS
    wgt_parts = [None] * CHUNKS
    logits_parts = [None] * CHUNKS
    logits_parts[0] = _tc_logits_chunk(x, weight, tpc, 0)
    for c in range(1, CHUNKS):
        logits_parts[c] = _tc_logits_chunk(x, weight, tpc, c)
        idx_parts[c - 1], wgt_parts[c - 1] = _sc_topk_chunk(logits_parts[c - 1], tpc)
    idx_parts[-1], wgt_parts[-1] = _sc_topk_chunk(logits_parts[-1], tpc)
    idx3 = jnp.concatenate(idx_parts, axis=0)
    wgt3 = jnp.concatenate(wgt_parts, axis=0)
    idx = idx3.transpose(0, 2, 1).reshape(t, TOPK)
    wgt = wgt3.transpose(0, 2, 1).reshape(t, TOPK)
    return idx, wgt
